# baseline (speedup 1.0000x reference)
import functools

import jax
import jax.numpy as jnp
from jax import lax
from jax.experimental import pallas as pl
from jax.experimental.pallas import tpu as pltpu
from jax.experimental.pallas import tpu_sc as plsc

EMB_SIZE = 1000000
EMB_DIM = 64
BATCH = 16384
NUM_CORES = 2
NUM_SUBCORES = 16
L = 16
NW = NUM_CORES * NUM_SUBCORES
NBLK = (EMB_SIZE + 127) // 128
BPT = (NBLK + NW - 1) // NW
CAP_U = 1024
CAP_V = 2048
NROWS = 3 * BATCH
DUMMY = NROWS + 127
NROWS_PAD = NROWS + 128
ISTAGE = 2048
CLIP = 1.0e10


def _sweep_body(pu_hbm, pv_hbm, nv_hbm, ut_hbm, vt_hbm, rows_hbm,
                istage, lu_slot, lu_idx, lv_slot, lv_idx,
                smat_u, smat_v, sweep, rowstage, sem, gsem):
    wid = lax.axis_index("s") * NUM_CORES + lax.axis_index("c")
    lo = wid * BPT
    hi = jnp.minimum(lo + BPT, NBLK)
    lane = lax.iota(jnp.int32, L)

    dummy16 = jnp.full((L,), DUMMY, jnp.int32)
    zero16 = jnp.zeros((L,), jnp.int32)

    def init_body(q, _):
        lv_slot[pl.ds(q * L, L)] = dummy16
        lv_idx[pl.ds(q * L, L)] = zero16
        return 0

    lax.fori_loop(0, CAP_V // L, init_body, 0)

    def init_body_u(q, _):
        lu_slot[pl.ds(q * L, L)] = dummy16
        lu_idx[pl.ds(q * L, L)] = zero16
        return 0

    lax.fori_loop(0, CAP_U // L, init_body_u, 0)

    def scan_array(idx_hbm, slot_base, cnt, slot_ref, idx_ref, cap):
        for k in range(BATCH // ISTAGE):
            pltpu.sync_copy(idx_hbm.at[pl.ds(k * ISTAGE, ISTAGE)], istage)

            def scan_body(q, cnt, k=k):
                i = istage[pl.ds(q * L, L)]
                blk = lax.shift_right_logical(i, 7)
                mask = jnp.logical_and(blk >= lo, blk < hi)
                n = jnp.sum(mask.astype(jnp.int32))

                @pl.when(n > 0)
                def _():
                    w = jnp.minimum(cnt, cap - L)
                    slots = slot_base + k * ISTAGE + q * L + lane
                    plsc.store_compressed(slot_ref.at[pl.ds(w, L)], slots, mask=mask)
                    plsc.store_compressed(idx_ref.at[pl.ds(w, L)], i, mask=mask)

                return cnt + n

            cnt = lax.fori_loop(0, ISTAGE // L, scan_body, cnt)
        return cnt

    cnt_u = scan_array(pu_hbm, 0, jnp.int32(0), lu_slot, lu_idx, CAP_U)
    cnt_v = scan_array(pv_hbm, BATCH, jnp.int32(0), lv_slot, lv_idx, CAP_V)
    cnt_v = scan_array(nv_hbm, 2 * BATCH, cnt_v, lv_slot, lv_idx, CAP_V)

    def smat_body_u(g, _):
        s = lu_slot[pl.ds(g * L, L)]
        plsc.store_scatter(smat_u, [jnp.full((L,), g, jnp.int32), lane], s)
        return 0

    lax.fori_loop(0, CAP_U // L, smat_body_u, 0)

    def smat_body_v(g, _):
        s = lv_slot[pl.ds(g * L, L)]
        plsc.store_scatter(smat_v, [jnp.full((L,), g, jnp.int32), lane], s)
        return 0

    lax.fori_loop(0, CAP_V // L, smat_body_v, 0)

    ng_u = lax.div(cnt_u + (L - 1), jnp.int32(L))
    ng_v = lax.div(cnt_v + (L - 1), jnp.int32(L))

    def extract_group(idx_ref, smat, g, blk_id, tab_sel):
        i = idx_ref[pl.ds(g * L, L)]
        blk = lax.shift_right_logical(i, 7)
        mask = blk == blk_id
        n = jnp.sum(mask.astype(jnp.int32))

        @pl.when(n > 0)
        def _():
            c = lax.bitwise_and(i, 127)

            def dloop(d, _, c=c, tab_sel=tab_sel):
                dd = jnp.full((L,), d, jnp.int32)
                vals = plsc.load_gather(sweep, [jnp.full((L,), tab_sel, jnp.int32), dd, c])
                plsc.store_scatter(rowstage, [lane, dd], vals)
                return 0

            lax.fori_loop(0, EMB_DIM, dloop, 0)
            pltpu.async_copy(rowstage, rows_hbm.at[smat.at[g]], sem).wait()

    def sweep_chunk(b, _):
        blk_id = lo + b
        off = pl.multiple_of(blk_id * 128, 128)
        cu = pltpu.async_copy(ut_hbm.at[:, pl.ds(off, 128)], sweep.at[0], gsem)
        cv = pltpu.async_copy(vt_hbm.at[:, pl.ds(off, 128)], sweep.at[1], gsem)
        cu.wait()
        cv.wait()

        def gu(g, _):
            extract_group(lu_idx, smat_u, g, blk_id, 0)
            return 0

        lax.fori_loop(0, ng_u, gu, 0)

        def gv(g, _):
            extract_group(lv_idx, smat_v, g, blk_id, 1)
            return 0

        lax.fori_loop(0, ng_v, gv, 0)
        return 0

    lax.fori_loop(0, hi - lo, sweep_chunk, 0)


_sweep = functools.partial(
    pl.kernel,
    out_type=jax.ShapeDtypeStruct((NROWS_PAD, 128), jnp.float32),
    mesh=plsc.VectorSubcoreMesh(
        core_axis_name="c", subcore_axis_name="s",
        num_cores=NUM_CORES, num_subcores=NUM_SUBCORES),
    compiler_params=pltpu.CompilerParams(needs_layout_passes=False),
    scratch_types=[
        pltpu.VMEM((ISTAGE,), jnp.int32),
        pltpu.VMEM((CAP_U,), jnp.int32),
        pltpu.VMEM((CAP_U,), jnp.int32),
        pltpu.VMEM((CAP_V,), jnp.int32),
        pltpu.VMEM((CAP_V,), jnp.int32),
        pltpu.VMEM((CAP_U // L, L), jnp.int32),
        pltpu.VMEM((CAP_V // L, L), jnp.int32),
        pltpu.VMEM((2, EMB_DIM, 128), jnp.float32),
        pltpu.VMEM((L, 128), jnp.float32),
        pltpu.SemaphoreType.DMA,
        pltpu.SemaphoreType.DMA,
    ],
)(_sweep_body)


def _dots_body(rows_hbm, out_hbm, ubuf, vbuf, nbuf, spmat, sem):
    wid = lax.axis_index("s") * NUM_CORES + lax.axis_index("c")
    base = wid * (BATCH // NW)
    lane = lax.iota(jnp.int32, L)
    for j in range(4):
        eb = base + j * 128
        cu = pltpu.async_copy(rows_hbm.at[pl.ds(eb, 128)], ubuf, sem)
        cv = pltpu.async_copy(rows_hbm.at[pl.ds(BATCH + eb, 128)], vbuf, sem)
        cn = pltpu.async_copy(rows_hbm.at[pl.ds(2 * BATCH + eb, 128)], nbuf, sem)
        cu.wait()
        cv.wait()
        cn.wait()
        for g in range(8):
            e = g * L + lane

            def dbody(d, carry, e=e):
                su, sn = carry
                dd = jnp.full((L,), d, jnp.int32)
                uu = plsc.load_gather(ubuf, [e, dd])
                vv = plsc.load_gather(vbuf, [e, dd])
                nn = plsc.load_gather(nbuf, [e, dd])
                return su + uu * vv, sn + uu * nn

            zero = jnp.zeros((L,), jnp.float32)
            su, sn = lax.fori_loop(0, EMB_DIM, dbody, (zero, zero))
            col = g * L + lane
            plsc.store_scatter(spmat, [jnp.full((L,), j, jnp.int32), col], su)
            plsc.store_scatter(spmat, [jnp.full((L,), 4 + j, jnp.int32), col], sn)
    pltpu.sync_copy(spmat, out_hbm.at[pl.ds(wid * 8, 8)])


_dots = functools.partial(
    pl.kernel,
    out_type=jax.ShapeDtypeStruct((NW * 8, 128), jnp.float32),
    mesh=plsc.VectorSubcoreMesh(
        core_axis_name="c", subcore_axis_name="s",
        num_cores=NUM_CORES, num_subcores=NUM_SUBCORES),
    compiler_params=pltpu.CompilerParams(needs_layout_passes=False),
    scratch_types=[
        pltpu.VMEM((128, 128), jnp.float32),
        pltpu.VMEM((128, 128), jnp.float32),
        pltpu.VMEM((128, 128), jnp.float32),
        pltpu.VMEM((8, 128), jnp.float32),
        pltpu.SemaphoreType.DMA,
    ],
)(_dots_body)


def _loss_body(s_ref, o_ref):
    x = s_ref[...]
    rid = lax.broadcasted_iota(jnp.int32, x.shape, 0)
    sgn = jnp.where((rid % 8) < 4, -1.0, 1.0).astype(jnp.float32)
    x = jnp.clip(x, -CLIP, CLIP)
    z = sgn * x
    loss = jnp.maximum(z, 0.0) + jnp.log1p(jnp.exp(-jnp.abs(z)))
    o_ref[...] = (jnp.sum(loss) * (1.0 / BATCH)).reshape(1, 1)


def kernel(pos_u, pos_v, neg_v, u_embeddings, v_embeddings):
    rows = _sweep(pos_u, pos_v, neg_v, u_embeddings.T, v_embeddings.T)
    scores = _dots(rows)
    out = pl.pallas_call(
        _loss_body,
        out_shape=jax.ShapeDtypeStruct((1, 1), jnp.float32),
    )(scores)
    return out[0, 0]
